# baseline (speedup 1.0000x reference)
import functools

import jax
import jax.numpy as jnp
from jax import lax
from jax.experimental import pallas as pl
from jax.experimental.pallas import tpu as pltpu
from jax.experimental.pallas import tpu_sc as plsc

_B, _L = 16384, 200
_V, _D = 1000, 10
_N = _B * _L
_NC, _NS = 2, 16
_NW = _NC * _NS
_PER_W = _N // _NW
_BPC = 2
_CHUNK = _BPC * _L
_BPW = _B // _NW
_NCHUNK = _BPW // _BPC
_STEPS = _CHUNK // 16


def _fold_table_tc(emb, w, b):
    def body(e_ref, w_ref, b_ref, o_ref):
        o_ref[...] = (
            jnp.dot(e_ref[...], w_ref[...].T, preferred_element_type=jnp.float32)
            + b_ref[...]
        )

    return pl.pallas_call(
        body,
        out_shape=jax.ShapeDtypeStruct((_V, _D), jnp.float32),
    )(emb, w, b.reshape(1, _D))


def _gather_sc(table_flat, idx_flat):
    mesh = plsc.VectorSubcoreMesh(core_axis_name="c", subcore_axis_name="s")

    @functools.partial(
        pl.kernel,
        mesh=mesh,
        out_type=jax.ShapeDtypeStruct((_B, _L, _D), jnp.float32),
        scratch_types=[
            pltpu.VMEM((_V * _D,), jnp.float32),
            pltpu.VMEM((_CHUNK,), jnp.int32),
            pltpu.VMEM((_BPC, _L, _D), jnp.float32),
        ],
        compiler_params=pltpu.CompilerParams(needs_layout_passes=False),
    )
    def k(table_hbm, idx_hbm, out_hbm, table_v, idx_v, out_v):
        wid = lax.axis_index("s") * _NC + lax.axis_index("c")
        pltpu.sync_copy(table_hbm, table_v)
        ii = lax.iota(jnp.int32, 16)
        base = wid * _BPW * _L

        def chunk_body(c, carry):
            off = base + c * _CHUNK
            pltpu.sync_copy(idx_hbm.at[pl.ds(off, _CHUNK)], idx_v)

            def jbody(j, carry2):
                iv = idx_v[pl.ds(j * 16, 16)]
                rb = iv * _D
                orow = ii + j * 16
                ob = orow // _L
                ol = orow % _L
                for dd in range(_D):
                    vals = plsc.load_gather(table_v, [rb + dd])
                    plsc.store_scatter(
                        out_v, [ob, ol, jnp.full((16,), dd, jnp.int32)], vals
                    )
                return carry2

            lax.fori_loop(0, _STEPS, jbody, 0)
            pltpu.sync_copy(
                out_v, out_hbm.at[pl.ds(wid * _BPW + c * _BPC, _BPC), :, :]
            )
            return carry

        lax.fori_loop(0, _NCHUNK, chunk_body, 0)

    return k(table_flat, idx_flat)


def kernel(x, embedding_weight, rnn_weight, rnn_bias):
    t = _fold_table_tc(embedding_weight, rnn_weight, rnn_bias)
    idx = x.reshape(-1).astype(jnp.int32)
    return _gather_sc(t.reshape(-1), idx)


# speedup vs baseline: 1.4053x; 1.4053x over previous
import functools

import jax
import jax.numpy as jnp
from jax import lax
from jax.experimental import pallas as pl
from jax.experimental.pallas import tpu as pltpu
from jax.experimental.pallas import tpu_sc as plsc

_B, _L = 16384, 200
_V, _D = 1000, 10
_N = _B * _L
_NC, _NS = 2, 16
_NW = _NC * _NS
_PER_W = _N // _NW
_BPC = 2
_CHUNK = _BPC * _L
_BPW = _B // _NW
_NCHUNK = _BPW // _BPC
_STEPS = _CHUNK // 16


def _fold_table_tc(emb, w, b):
    def body(e_ref, w_ref, b_ref, o_ref):
        o_ref[...] = (
            jnp.dot(e_ref[...], w_ref[...].T, preferred_element_type=jnp.float32)
            + b_ref[...]
        )

    return pl.pallas_call(
        body,
        out_shape=jax.ShapeDtypeStruct((_V, _D), jnp.float32),
    )(emb, w, b.reshape(1, _D))


def _gather_sc(table_flat, idx_flat):
    mesh = plsc.VectorSubcoreMesh(core_axis_name="c", subcore_axis_name="s")

    @functools.partial(
        pl.kernel,
        mesh=mesh,
        out_type=jax.ShapeDtypeStruct((_B, _L, _D), jnp.float32),
        scratch_types=[
            pltpu.VMEM((_V * _D,), jnp.float32),
            pltpu.VMEM((_CHUNK,), jnp.int32),
            pltpu.VMEM((_CHUNK,), jnp.int32),
            pltpu.VMEM((_BPC, _L, _D), jnp.float32),
            pltpu.VMEM((_BPC, _L, _D), jnp.float32),
            pltpu.SemaphoreType.DMA,
            pltpu.SemaphoreType.DMA,
            pltpu.SemaphoreType.DMA,
            pltpu.SemaphoreType.DMA,
        ],
        compiler_params=pltpu.CompilerParams(needs_layout_passes=False),
    )
    def k(table_hbm, idx_hbm, out_hbm, table_v, idx_v0, idx_v1,
          out_v0, out_v1, si0, si1, so0, so1):
        wid = lax.axis_index("s") * _NC + lax.axis_index("c")
        pltpu.sync_copy(table_hbm, table_v)
        ii = lax.iota(jnp.int32, 16)
        base = wid * _BPW * _L
        brow = wid * _BPW
        idx_bufs = (idx_v0, idx_v1)
        out_bufs = (out_v0, out_v1)
        sis = (si0, si1)
        sos = (so0, so1)

        def idx_slice(c):
            return idx_hbm.at[pl.ds(base + c * _CHUNK, _CHUNK)]

        def out_slice(c):
            return out_hbm.at[pl.ds(brow + c * _BPC, _BPC), :, :]

        def compute(idx_v, out_v):
            def jbody(j, carry2):
                iv = idx_v[pl.ds(j * 16, 16)]
                rb = iv * _D
                orow = ii + j * 16
                ob = orow // _L
                ol = orow % _L
                for dd in range(_D):
                    vals = plsc.load_gather(table_v, [rb + dd])
                    plsc.store_scatter(
                        out_v, [ob, ol, jnp.full((16,), dd, jnp.int32)], vals
                    )
                return carry2

            lax.fori_loop(0, _STEPS, jbody, 0)

        pltpu.async_copy(idx_slice(0), idx_v0, si0)

        def pair_body(o, carry):
            for par in range(2):
                c = o * 2 + par
                pltpu.make_async_copy(idx_slice(c), idx_bufs[par], sis[par]).wait()
                nxt = c + 1

                @pl.when(nxt < _NCHUNK)
                def _():
                    pltpu.async_copy(
                        idx_slice(nxt), idx_bufs[1 - par], sis[1 - par]
                    )

                @pl.when(o > 0)
                def _():
                    pltpu.make_async_copy(
                        out_bufs[par], out_slice(c - 2), sos[par]
                    ).wait()

                compute(idx_bufs[par], out_bufs[par])
                pltpu.async_copy(out_bufs[par], out_slice(c), sos[par])
            return carry

        lax.fori_loop(0, _NCHUNK // 2, pair_body, 0)
        pltpu.make_async_copy(out_v0, out_slice(_NCHUNK - 2), so0).wait()
        pltpu.make_async_copy(out_v1, out_slice(_NCHUNK - 1), so1).wait()

    return k(table_flat, idx_flat)


def kernel(x, embedding_weight, rnn_weight, rnn_bias):
    t = _fold_table_tc(embedding_weight, rnn_weight, rnn_bias)
    idx = x.reshape(-1).astype(jnp.int32)
    return _gather_sc(t.reshape(-1), idx)


# speedup vs baseline: 1.5870x; 1.1293x over previous
import functools

import jax
import jax.numpy as jnp
from jax import lax
from jax.experimental import pallas as pl
from jax.experimental.pallas import tpu as pltpu
from jax.experimental.pallas import tpu_sc as plsc

_B, _L = 16384, 200
_V, _D = 1000, 10
_N = _B * _L
_NC, _NS = 2, 16
_NW = _NC * _NS
_PER_W = _N // _NW
_BPC = 2
_CHUNK = _BPC * _L
_BPW = _B // _NW
_NCHUNK = _BPW // _BPC
_STEPS = _CHUNK // 16


def _fold_table_tc(emb, w, b):
    def body(e_ref, w_ref, b_ref, o_ref):
        o_ref[...] = (
            jnp.dot(e_ref[...], w_ref[...].T, preferred_element_type=jnp.float32)
            + b_ref[...]
        )

    return pl.pallas_call(
        body,
        out_shape=jax.ShapeDtypeStruct((_V, _D), jnp.float32),
    )(emb, w, b.reshape(1, _D))


def _gather_sc(table_flat, idx_flat):
    mesh = plsc.VectorSubcoreMesh(core_axis_name="c", subcore_axis_name="s")

    @functools.partial(
        pl.kernel,
        mesh=mesh,
        out_type=jax.ShapeDtypeStruct((_B, _L, _D), jnp.float32),
        scratch_types=[
            pltpu.VMEM((_V * _D,), jnp.float32),
            pltpu.VMEM((_CHUNK,), jnp.int32),
            pltpu.VMEM((_CHUNK,), jnp.int32),
            pltpu.VMEM((_BPC, _L, _D), jnp.float32),
            pltpu.VMEM((_BPC, _L, _D), jnp.float32),
            pltpu.SemaphoreType.DMA,
            pltpu.SemaphoreType.DMA,
            pltpu.SemaphoreType.DMA,
            pltpu.SemaphoreType.DMA,
        ],
        compiler_params=pltpu.CompilerParams(needs_layout_passes=False),
    )
    def k(table_hbm, idx_hbm, out_hbm, table_v, idx_v0, idx_v1,
          out_v0, out_v1, si0, si1, so0, so1):
        wid = lax.axis_index("s") * _NC + lax.axis_index("c")
        pltpu.sync_copy(table_hbm, table_v)
        ii = lax.iota(jnp.int32, 16)
        base = wid * _BPW * _L
        brow = wid * _BPW
        idx_bufs = (idx_v0, idx_v1)
        out_bufs = (out_v0, out_v1)
        sis = (si0, si1)
        sos = (so0, so1)

        def idx_slice(c):
            return idx_hbm.at[pl.ds(base + c * _CHUNK, _CHUNK)]

        def out_slice(c):
            return out_hbm.at[pl.ds(brow + c * _BPC, _BPC), :, :]

        def compute(idx_v, out_v):
            def jbody(j, carry2):
                iv = idx_v[pl.ds(j * 16, 16)]
                rb = iv * _D
                orow = ii + j * 16
                ob = jnp.where(orow >= _L, 1, 0).astype(jnp.int32)
                ol = orow - ob * _L
                vals = [plsc.load_gather(table_v, [rb + dd]) for dd in range(_D)]
                for dd in range(_D):
                    plsc.store_scatter(
                        out_v, [ob, ol, jnp.full((16,), dd, jnp.int32)], vals[dd]
                    )
                return carry2

            lax.fori_loop(0, _STEPS, jbody, 0)

        pltpu.async_copy(idx_slice(0), idx_v0, si0)

        def pair_body(o, carry):
            for par in range(2):
                c = o * 2 + par
                pltpu.make_async_copy(idx_slice(c), idx_bufs[par], sis[par]).wait()
                nxt = c + 1

                @pl.when(nxt < _NCHUNK)
                def _():
                    pltpu.async_copy(
                        idx_slice(nxt), idx_bufs[1 - par], sis[1 - par]
                    )

                @pl.when(o > 0)
                def _():
                    pltpu.make_async_copy(
                        out_bufs[par], out_slice(c - 2), sos[par]
                    ).wait()

                compute(idx_bufs[par], out_bufs[par])
                pltpu.async_copy(out_bufs[par], out_slice(c), sos[par])
            return carry

        lax.fori_loop(0, _NCHUNK // 2, pair_body, 0)
        pltpu.make_async_copy(out_v0, out_slice(_NCHUNK - 2), so0).wait()
        pltpu.make_async_copy(out_v1, out_slice(_NCHUNK - 1), so1).wait()

    return k(table_flat, idx_flat)


def kernel(x, embedding_weight, rnn_weight, rnn_bias):
    t = _fold_table_tc(embedding_weight, rnn_weight, rnn_bias)
    idx = x.reshape(-1).astype(jnp.int32)
    return _gather_sc(t.reshape(-1), idx)


# speedup vs baseline: 1.6042x; 1.0108x over previous
import functools

import jax
import jax.numpy as jnp
from jax import lax
from jax.experimental import pallas as pl
from jax.experimental.pallas import tpu as pltpu
from jax.experimental.pallas import tpu_sc as plsc

_B, _L = 16384, 200
_V, _D = 1000, 10
_NC, _NS = 2, 16
_NW = _NC * _NS
_BPW = _B // _NW
_IPC = 8
_NIC = _BPW // _IPC
_BPC = 2
_SUBS = _IPC // _BPC
_STEPS = _BPC * _L // 16


def _fold_table_tc(emb, w, b):
    def body(e_ref, w_ref, b_ref, o_ref):
        o_ref[...] = (
            jnp.dot(e_ref[...], w_ref[...].T, preferred_element_type=jnp.float32)
            + b_ref[...]
        )

    return pl.pallas_call(
        body,
        out_shape=jax.ShapeDtypeStruct((_V, _D), jnp.float32),
    )(emb, w, b.reshape(1, _D))


def _gather_sc(table_flat, x):
    mesh = plsc.VectorSubcoreMesh(core_axis_name="c", subcore_axis_name="s")

    @functools.partial(
        pl.kernel,
        mesh=mesh,
        out_type=jax.ShapeDtypeStruct((_B, _L, _D), jnp.float32),
        scratch_types=[
            pltpu.VMEM((_V * _D,), jnp.float32),
            pltpu.VMEM((_IPC, _L), jnp.int32),
            pltpu.VMEM((_IPC, _L), jnp.int32),
            pltpu.VMEM((_BPC, _L, _D), jnp.float32),
            pltpu.VMEM((_BPC, _L, _D), jnp.float32),
            pltpu.SemaphoreType.DMA,
            pltpu.SemaphoreType.DMA,
            pltpu.SemaphoreType.DMA,
            pltpu.SemaphoreType.DMA,
        ],
        compiler_params=pltpu.CompilerParams(needs_layout_passes=False),
    )
    def k(table_hbm, x_hbm, out_hbm, table_v, x_v0, x_v1,
          out_v0, out_v1, si0, si1, so0, so1):
        wid = lax.axis_index("s") * _NC + lax.axis_index("c")
        pltpu.sync_copy(table_hbm, table_v)
        ii = lax.iota(jnp.int32, 16)
        brow = wid * _BPW
        x_bufs = (x_v0, x_v1)
        out_bufs = (out_v0, out_v1)
        sis = (si0, si1)
        sos = (so0, so1)

        def x_slice(g):
            return x_hbm.at[pl.ds(brow + g * _IPC, _IPC), :]

        def out_slice(c):
            return out_hbm.at[pl.ds(brow + c * _BPC, _BPC), :, :]

        def compute(x_v, out_v, sub):
            def jbody(j, carry2):
                orow = ii + j * 16
                ob = jnp.where(orow >= _L, 1, 0).astype(jnp.int32)
                ol = orow - ob * _L
                iv = plsc.load_gather(x_v, [ob + sub * _BPC, ol])
                rb = iv * _D
                vals = [plsc.load_gather(table_v, [rb + dd]) for dd in range(_D)]
                for dd in range(_D):
                    plsc.store_scatter(
                        out_v, [ob, ol, jnp.full((16,), dd, jnp.int32)], vals[dd]
                    )
                return carry2

            lax.fori_loop(0, _STEPS, jbody, 0)

        pltpu.async_copy(x_slice(0), x_v0, si0)

        def pair_body(o, carry):
            for par in range(2):
                g = o * 2 + par
                pltpu.make_async_copy(x_slice(g), x_bufs[par], sis[par]).wait()
                nxt = g + 1

                @pl.when(nxt < _NIC)
                def _():
                    pltpu.async_copy(x_slice(nxt), x_bufs[1 - par], sis[1 - par])

                for sub in range(_SUBS):
                    c = g * _SUBS + sub
                    po = sub % 2
                    if sub < 2:
                        @pl.when(g > 0)
                        def _():
                            pltpu.make_async_copy(
                                out_bufs[po], out_slice(c - 2), sos[po]
                            ).wait()
                    else:
                        pltpu.make_async_copy(
                            out_bufs[po], out_slice(c - 2), sos[po]
                        ).wait()
                    compute(x_bufs[par], out_bufs[po], sub)
                    pltpu.async_copy(out_bufs[po], out_slice(c), sos[po])
            return carry

        lax.fori_loop(0, _NIC // 2, pair_body, 0)
        nc = _NIC * _SUBS
        pltpu.make_async_copy(out_v0, out_slice(nc - 2), so0).wait()
        pltpu.make_async_copy(out_v1, out_slice(nc - 1), so1).wait()

    return k(table_flat, x)


def kernel(x, embedding_weight, rnn_weight, rnn_bias):
    t = _fold_table_tc(embedding_weight, rnn_weight, rnn_bias)
    return _gather_sc(t.reshape(-1), x.astype(jnp.int32))


# speedup vs baseline: 19.9487x; 12.4353x over previous
import functools

import jax
import jax.numpy as jnp
from jax import lax
from jax.experimental import pallas as pl
from jax.experimental.pallas import tpu as pltpu
from jax.experimental.pallas import tpu_sc as plsc

_B, _L = 16384, 200
_V, _D = 1000, 10
_TSW = 7
_TS = 2 * _TSW
_NC, _NS = 2, 16
_NW = _NC * _NS
_BPW = _B // _NW
_BBS = 128
_NBB = _BPW // _BBS
_LBS = 8
_NLB = _L // _LBS


def _fold_table_tc(emb, w_pad, b_pad):
    def body(e_ref, w_ref, b_ref, o_ref):
        o_ref[...] = (
            jnp.dot(e_ref[...], w_ref[...].T, preferred_element_type=jnp.float32)
            + b_ref[...]
        )

    return pl.pallas_call(
        body,
        out_shape=jax.ShapeDtypeStruct((_V, _TS), jnp.float32),
    )(emb, w_pad, b_pad.reshape(1, _TS))


def _gather_sc(table_flat, xt):
    mesh = plsc.VectorSubcoreMesh(core_axis_name="c", subcore_axis_name="s")

    @functools.partial(
        pl.kernel,
        mesh=mesh,
        out_type=jax.ShapeDtypeStruct((_D, _L, _B), jnp.float32),
        scratch_types=[
            pltpu.VMEM((_V * _TSW,), jnp.int32),
            pltpu.VMEM((_L, _BBS), jnp.int32),
            pltpu.VMEM((_L, _BBS), jnp.int32),
            pltpu.VMEM((_D, _LBS, _BBS), jnp.float32),
            pltpu.VMEM((_D, _LBS, _BBS), jnp.float32),
            pltpu.SemaphoreType.DMA,
            pltpu.SemaphoreType.DMA,
            pltpu.SemaphoreType.DMA,
            pltpu.SemaphoreType.DMA,
        ],
        compiler_params=pltpu.CompilerParams(needs_layout_passes=False),
    )
    def k(table_hbm, xt_hbm, out_hbm, table_v, x_v0, x_v1,
          out_v0, out_v1, si0, si1, so0, so1):
        wid = lax.axis_index("s") * _NC + lax.axis_index("c")
        bbase = wid * _BPW
        x_bufs = (x_v0, x_v1)
        out_bufs = (out_v0, out_v1)
        sis = (si0, si1)
        sos = (so0, so1)

        def x_slice(bb):
            return xt_hbm.at[:, pl.ds(bbase + bb * _BBS, _BBS)]

        def out_slice(lb, b0):
            return out_hbm.at[:, pl.ds(lb * _LBS, _LBS), pl.ds(b0, _BBS)]

        def compute(x_v, out_v, lb):
            l0 = lb * _LBS
            ngrp = _BBS // 16

            def gathers(lrow, bg):
                iv = x_v[lrow, pl.ds(bg * 16, 16)]
                rb = iv * _TSW
                return [
                    plsc.load_gather(table_v, [rb + j]) for j in range(_D // 2)
                ]

            def stores(ws, lo, bg):
                for j in range(_D // 2):
                    lohi = plsc.unpack(
                        plsc.bitcast(ws[j], jnp.bfloat16),
                        format=plsc.PackFormat.INTERLEAVED,
                    )
                    out_v[2 * j, lo, pl.ds(bg * 16, 16)] = lohi[0]
                    out_v[2 * j + 1, lo, pl.ds(bg * 16, 16)] = lohi[1]

            def lobody(lo, carry):
                lrow = l0 + lo
                ws = gathers(lrow, 0)
                for bg in range(1, ngrp):
                    ws_next = gathers(lrow, bg)
                    stores(ws, lo, bg - 1)
                    ws = ws_next
                stores(ws, lo, ngrp - 1)
                return carry

            lax.fori_loop(0, _LBS, lobody, 0)

        pltpu.async_copy(x_slice(0), x_v0, si0)
        pltpu.sync_copy(table_hbm, table_v)

        for bb in range(_NBB):
            xp = bb % 2
            pltpu.make_async_copy(x_slice(bb), x_bufs[xp], sis[xp]).wait()
            if bb + 1 < _NBB:
                pltpu.async_copy(
                    x_slice(bb + 1), x_bufs[1 - xp], sis[1 - xp]
                )
            b0 = bbase + bb * _BBS
            b0_prev = bbase + (bb - 1) * _BBS

            def lpair(o, carry, bb=bb, b0=b0, b0_prev=b0_prev):
                for par2 in range(2):
                    lb = o * 2 + par2
                    po = (bb + par2) % 2
                    @pl.when(o >= 1)
                    def _():
                        pltpu.make_async_copy(
                            out_bufs[po], out_slice(lb - 2, b0), sos[po]
                        ).wait()

                    if bb > 0:
                        @pl.when(o == 0)
                        def _():
                            pltpu.make_async_copy(
                                out_bufs[po],
                                out_slice(23 + par2, b0_prev),
                                sos[po],
                            ).wait()

                    compute(x_bufs[xp], out_bufs[po], lb)
                    pltpu.async_copy(out_bufs[po], out_slice(lb, b0), sos[po])
                return carry

            lax.fori_loop(0, (_NLB - 1) // 2, lpair, 0)
            po = bb % 2
            pltpu.make_async_copy(
                out_bufs[po], out_slice(_NLB - 3, b0), sos[po]
            ).wait()
            compute(x_bufs[xp], out_bufs[po], _NLB - 1)
            pltpu.async_copy(out_bufs[po], out_slice(_NLB - 1, b0), sos[po])

        b0_last = bbase + (_NBB - 1) * _BBS
        pltpu.make_async_copy(
            out_bufs[(_NBB - 1 + 1) % 2], out_slice(_NLB - 2, b0_last),
            sos[(_NBB - 1 + 1) % 2],
        ).wait()
        pltpu.make_async_copy(
            out_bufs[(_NBB - 1) % 2], out_slice(_NLB - 1, b0_last),
            sos[(_NBB - 1) % 2],
        ).wait()

    return k(table_flat, xt)


def kernel(x, embedding_weight, rnn_weight, rnn_bias):
    w_pad = jnp.concatenate(
        [rnn_weight, jnp.zeros((_TS - _D, _D), jnp.float32)], axis=0
    )
    b_pad = jnp.concatenate([rnn_bias, jnp.zeros((_TS - _D,), jnp.float32)])
    t = _fold_table_tc(embedding_weight, w_pad, b_pad)
    tb = t.astype(jnp.bfloat16).reshape(_V, _TSW, 2)
    tw = jax.lax.bitcast_convert_type(tb, jnp.int32)
    xt = x.T.astype(jnp.int32)
    out_t = _gather_sc(tw.reshape(-1), xt)
    return out_t.transpose(2, 1, 0)
